# baseline (speedup 1.0000x reference)
import functools

import jax
import jax.numpy as jnp
from jax import lax
from jax.experimental import pallas as pl
from jax.experimental.pallas import tpu as pltpu
from jax.experimental.pallas import tpu_sc as plsc

N_NODES = 10000
N_EDGES = 320000
D_FEAT = 128
D_EDGE = 16

NC = 2
NS = 16
NW = NC * NS
CHUNK = 56
GSZ = 38
NG = 8
G0 = 38
G1 = 8
T0 = NG * G0 * CHUNK
T1 = NG * G1 * CHUNK
CORE0_EDGES = NS * T0
E_PAD = NS * (T0 + T1)
N_PAD = 10240
ROWS_PT = N_PAD // NS
ZROWS = 128


def _gate_body(ea_ref, we_ref, out_ref):
    out_ref[...] = jax.lax.dot_general(
        ea_ref[...], we_ref[...], (((1,), (0,)), ((), ())),
        preferred_element_type=jnp.float32)


def _gate_tc(ea_pad, w_edge):
    blk = 4096
    grid = E_PAD // blk
    return pl.pallas_call(
        _gate_body,
        grid=(grid,),
        in_specs=[
            pl.BlockSpec((blk, D_EDGE), lambda i: (i, 0)),
            pl.BlockSpec((D_EDGE, D_FEAT), lambda i: (0, 0)),
        ],
        out_specs=pl.BlockSpec((blk, D_FEAT), lambda i: (i, 0)),
        out_shape=jax.ShapeDtypeStruct((E_PAD, D_FEAT), jnp.float32),
    )(ea_pad, w_edge)


def _msg_body(x_hbm, src_hbm, dst_hbm, gate_hbm, out_hbm,
              src0, dst0, src1, dst1, rows0, rows1, gbuf0, gbuf1, hacc_sh,
              semg0, semg1, seml0, seml1, sems0, sems1, semi):
    cid = lax.axis_index("c")
    sid = lax.axis_index("s")
    wid = cid * NS + sid
    gc = jnp.where(cid == 0, G0, G1)
    npair = jnp.where(cid == 0, G0 // 2, G1 // 2)
    start = jnp.where(cid == 0, sid * T0, CORE0_EDGES + sid * T1)

    idx = ((src0, dst0), (src1, dst1))
    rows = (rows0, rows1)
    gbuf = (gbuf0, gbuf1)
    semg = (semg0, semg1)
    seml = (seml0, seml1)
    sems = (sems0, sems1)

    def _zrow(r, _):
        for b in range(D_FEAT // 16):
            rows0[r, pl.ds(b * 16, 16)] = jnp.zeros((16,), jnp.float32)
        return 0
    lax.fori_loop(0, CHUNK, _zrow, 0)
    for z in range(ROWS_PT // CHUNK):
        pltpu.sync_copy(rows0,
                        hacc_sh.at[pl.ds(sid * ROWS_PT + z * CHUNK, CHUNK)])
    plsc.subcore_barrier()

    def stage_idx(g):
        b = g % 2
        pltpu.async_copy(src_hbm.at[wid, g], idx[b][0], semi)
        pltpu.async_copy(dst_hbm.at[wid, g], idx[b][1], semi)

    def wait_idx(g):
        b = g % 2
        pltpu.make_async_copy(src_hbm.at[wid, 0], idx[b][0], semi).wait()
        pltpu.make_async_copy(src_hbm.at[wid, 0], idx[b][1], semi).wait()

    def issue_loads(p, ib, cj, g):
        off = start + (g * gc + cj) * CHUNK
        pltpu.async_copy(x_hbm.at[idx[ib][0].at[cj]], rows[p], semg[p])
        pltpu.async_copy(gate_hbm.at[pl.ds(off, CHUNK)], gbuf[p], seml[p])

    def wait_loads(p):
        pltpu.make_async_copy(gate_hbm.at[pl.ds(0, CHUNK)],
                              gbuf[p], seml[p]).wait()
        pltpu.make_async_copy(x_hbm.at[pl.ds(0, CHUNK)],
                              rows[p], semg[p]).wait()

    def compute(p):
        rp, gp = rows[p], gbuf[p]

        def _edge(e, _):
            for b in range(D_FEAT // 16):
                sl = pl.ds(b * 16, 16)
                gp[e, sl] = gp[e, sl] * rp[e, sl]
            return 0
        lax.fori_loop(0, CHUNK, _edge, 0)

    def issue_scatter(p, ib, cj):
        pltpu.async_copy(gbuf[p], hacc_sh.at[idx[ib][1].at[cj]], sems[p],
                         add=True)

    def wait_scatter(p):
        pltpu.make_async_copy(gbuf[p], hacc_sh.at[pl.ds(0, CHUNK)],
                              sems[p]).wait()

    stage_idx(0)
    wait_idx(0)
    issue_loads(0, 0, 0, 0)
    issue_loads(1, 0, 1, 0)

    for g in range(NG):
        ib = g % 2
        if g + 1 < NG:
            stage_idx(g + 1)

        def _pair(i, _, ib=ib, g=g):
            a = 2 * i
            wait_loads(0)
            compute(0)
            issue_scatter(0, ib, a)
            wait_loads(1)
            compute(1)
            issue_scatter(1, ib, a + 1)
            wait_scatter(0)
            issue_loads(0, ib, a + 2, g)
            wait_scatter(1)
            issue_loads(1, ib, a + 3, g)
            return 0
        lax.fori_loop(0, npair - 1, _pair, 0)

        a = 2 * npair - 2
        wait_loads(0)
        compute(0)
        issue_scatter(0, ib, a)
        wait_loads(1)
        compute(1)
        issue_scatter(1, ib, a + 1)
        if g + 1 < NG:
            wait_idx(g + 1)
            nb = (g + 1) % 2
            wait_scatter(0)
            issue_loads(0, nb, 0, g + 1)
            wait_scatter(1)
            issue_loads(1, nb, 1, g + 1)
        else:
            wait_scatter(0)
            wait_scatter(1)

    plsc.subcore_barrier()

    pltpu.sync_copy(hacc_sh.at[pl.ds(sid * ROWS_PT, ROWS_PT)],
                    out_hbm.at[cid, pl.ds(sid * ROWS_PT, ROWS_PT)])


def _msg_sc(x, src_t, dst_t, gate):
    mesh = plsc.VectorSubcoreMesh(core_axis_name="c", subcore_axis_name="s")
    f = pl.kernel(
        _msg_body,
        mesh=mesh,
        out_type=jax.ShapeDtypeStruct((NC, N_PAD, D_FEAT), jnp.float32),
        scratch_types=[
            pltpu.VMEM((GSZ, CHUNK), jnp.int32),
            pltpu.VMEM((GSZ, CHUNK), jnp.int32),
            pltpu.VMEM((GSZ, CHUNK), jnp.int32),
            pltpu.VMEM((GSZ, CHUNK), jnp.int32),
            pltpu.VMEM((CHUNK, D_FEAT), jnp.float32),
            pltpu.VMEM((CHUNK, D_FEAT), jnp.float32),
            pltpu.VMEM((CHUNK, D_FEAT), jnp.float32),
            pltpu.VMEM((CHUNK, D_FEAT), jnp.float32),
            pltpu.VMEM_SHARED((N_PAD, D_FEAT), jnp.float32),
            pltpu.SemaphoreType.DMA,
            pltpu.SemaphoreType.DMA,
            pltpu.SemaphoreType.DMA,
            pltpu.SemaphoreType.DMA,
            pltpu.SemaphoreType.DMA,
            pltpu.SemaphoreType.DMA,
            pltpu.SemaphoreType.DMA,
        ],
    )
    return f(x, src_t, dst_t, gate)


def _readout_body(hp_ref, wh_ref, wo_ref, sp_ref, cw_ref, out_ref):
    i = pl.program_id(0)
    h = hp_ref[0] + hp_ref[1]
    z = jax.lax.dot_general(h, wh_ref[...], (((1,), (0,)), ((), ())),
                            preferred_element_type=jnp.float32)
    s = z / (1.0 + jnp.exp(-z))
    e_sum = jnp.sum(s * wo_ref[...])
    sp = sp_ref[0, 0, :]
    comp = jnp.float32(0.0)
    for k in range(4):
        comp += cw_ref[0, k] * jnp.sum(jnp.where(sp == k, 1.0, 0.0))

    @pl.when(i == 0)
    def _():
        out_ref[0, 0] = 0.0

    out_ref[0, 0] += e_sum + comp


def _readout_tc(hpart, w_hidden, w_out2d, species3d, comp2d):
    blk = 2048
    grid = N_PAD // blk
    return pl.pallas_call(
        _readout_body,
        grid=(grid,),
        in_specs=[
            pl.BlockSpec((NC, blk, D_FEAT), lambda i: (0, i, 0)),
            pl.BlockSpec((D_FEAT, D_FEAT), lambda i: (0, 0)),
            pl.BlockSpec((1, D_FEAT), lambda i: (0, 0)),
            pl.BlockSpec((1, 1, blk), lambda i: (i, 0, 0)),
            pl.BlockSpec((1, 8), lambda i: (0, 0)),
        ],
        out_specs=pl.BlockSpec((1, 1), lambda i: (0, 0),
                               memory_space=pltpu.SMEM),
        out_shape=jax.ShapeDtypeStruct((1, 1), jnp.float32),
    )(hpart, w_hidden, w_out2d, species3d, comp2d)


def kernel(x, edge_index, edge_attr, species, W_edge, W_hidden, w_out, comp_w):
    pad = E_PAD - N_EDGES
    ea_pad = jnp.pad(edge_attr, ((0, pad), (0, 0)))
    def _tile_idx(flat):
        c0 = flat[:CORE0_EDGES].reshape(NS, NG, G0, CHUNK)
        c1 = flat[CORE0_EDGES:].reshape(NS, NG, G1, CHUNK)
        c1 = jnp.pad(c1, ((0, 0), (0, 0), (0, G0 - G1), (0, 0)))
        return jnp.concatenate([c0, c1], axis=0)

    src_t = _tile_idx(jnp.pad(edge_index[0], (0, pad)))
    dst_t = _tile_idx(jnp.pad(edge_index[1], (0, pad)))

    gate = _gate_tc(ea_pad, W_edge)
    hpart = _msg_sc(x, src_t, dst_t, gate)

    blk = 2048
    species_pad = jnp.pad(species, (0, N_PAD - N_NODES), constant_values=4)
    total = _readout_tc(
        hpart, W_hidden,
        w_out.reshape(1, D_FEAT),
        species_pad.reshape(N_PAD // blk, 1, blk),
        jnp.pad(comp_w, (0, 4)).reshape(1, 8),
    )
    return total[0, 0]


# speedup vs baseline: 1.1036x; 1.1036x over previous
import functools

import jax
import jax.numpy as jnp
from jax import lax
from jax.experimental import pallas as pl
from jax.experimental.pallas import tpu as pltpu
from jax.experimental.pallas import tpu_sc as plsc

N_NODES = 10000
N_EDGES = 320000
D_FEAT = 128
D_EDGE = 16

NC = 2
NS = 16
NW = NC * NS
CHUNK = 64
GSZ = 30
NG = 8
G0 = 30
G1 = 10
T0 = NG * G0 * CHUNK
T1 = NG * G1 * CHUNK
CORE0_EDGES = NS * T0
E_PAD = NS * (T0 + T1)
N_PAD = 10240
ROWS_PT = N_PAD // NS
ZROWS = 128


def _gate_body(ea_ref, we_ref, out_ref):
    out_ref[...] = jax.lax.dot_general(
        ea_ref[...], we_ref[...], (((1,), (0,)), ((), ())),
        preferred_element_type=jnp.float32)


def _gate_tc(ea_pad, w_edge):
    blk = 4096
    grid = E_PAD // blk
    return pl.pallas_call(
        _gate_body,
        grid=(grid,),
        in_specs=[
            pl.BlockSpec((blk, D_EDGE), lambda i: (i, 0)),
            pl.BlockSpec((D_EDGE, D_FEAT), lambda i: (0, 0)),
        ],
        out_specs=pl.BlockSpec((blk, D_FEAT), lambda i: (i, 0)),
        out_shape=jax.ShapeDtypeStruct((E_PAD, D_FEAT), jnp.float32),
    )(ea_pad, w_edge)


def _msg_body(x_hbm, src_hbm, dst_hbm, gate_hbm, out_hbm,
              src0, dst0, src1, dst1, rows0, rows1, gbuf0, gbuf1, hacc_sh,
              semg0, semg1, seml0, seml1, sems0, sems1, semi):
    cid = lax.axis_index("c")
    sid = lax.axis_index("s")
    wid = cid * NS + sid
    gc = jnp.where(cid == 0, G0, G1)
    npair = jnp.where(cid == 0, G0 // 2, G1 // 2)
    start = jnp.where(cid == 0, sid * T0, CORE0_EDGES + sid * T1)

    idx = ((src0, dst0), (src1, dst1))
    rows = (rows0, rows1)
    gbuf = (gbuf0, gbuf1)
    semg = (semg0, semg1)
    seml = (seml0, seml1)
    sems = (sems0, sems1)

    def _zrow(r, _):
        for b in range(D_FEAT // 16):
            rows0[r, pl.ds(b * 16, 16)] = jnp.zeros((16,), jnp.float32)
        return 0
    lax.fori_loop(0, CHUNK, _zrow, 0)
    for z in range(ROWS_PT // CHUNK):
        pltpu.sync_copy(rows0,
                        hacc_sh.at[pl.ds(sid * ROWS_PT + z * CHUNK, CHUNK)])
    plsc.subcore_barrier()

    def stage_idx(g):
        b = g % 2
        pltpu.async_copy(src_hbm.at[wid, g], idx[b][0], semi)
        pltpu.async_copy(dst_hbm.at[wid, g], idx[b][1], semi)

    def wait_idx(g):
        b = g % 2
        pltpu.make_async_copy(src_hbm.at[wid, 0], idx[b][0], semi).wait()
        pltpu.make_async_copy(src_hbm.at[wid, 0], idx[b][1], semi).wait()

    def issue_loads(p, ib, cj, g):
        off = start + (g * gc + cj) * CHUNK
        pltpu.async_copy(x_hbm.at[idx[ib][0].at[cj]], rows[p], semg[p])
        pltpu.async_copy(gate_hbm.at[pl.ds(off, CHUNK)], gbuf[p], seml[p])

    def wait_loads(p):
        pltpu.make_async_copy(gate_hbm.at[pl.ds(0, CHUNK)],
                              gbuf[p], seml[p]).wait()
        pltpu.make_async_copy(x_hbm.at[pl.ds(0, CHUNK)],
                              rows[p], semg[p]).wait()

    def compute(p):
        rp, gp = rows[p], gbuf[p]

        def _edge(e, _):
            for b in range(D_FEAT // 16):
                sl = pl.ds(b * 16, 16)
                gp[e, sl] = gp[e, sl] * rp[e, sl]
            return 0
        lax.fori_loop(0, CHUNK, _edge, 0)

    def issue_scatter(p, ib, cj):
        pltpu.async_copy(gbuf[p], hacc_sh.at[idx[ib][1].at[cj]], sems[p],
                         add=True)

    def wait_scatter(p):
        pltpu.make_async_copy(gbuf[p], hacc_sh.at[pl.ds(0, CHUNK)],
                              sems[p]).wait()

    stage_idx(0)
    wait_idx(0)
    issue_loads(0, 0, 0, 0)
    issue_loads(1, 0, 1, 0)

    for g in range(NG):
        ib = g % 2
        if g + 1 < NG:
            stage_idx(g + 1)

        def _pair(i, _, ib=ib, g=g):
            a = 2 * i
            wait_loads(0)
            compute(0)
            issue_scatter(0, ib, a)
            wait_loads(1)
            compute(1)
            issue_scatter(1, ib, a + 1)
            wait_scatter(0)
            issue_loads(0, ib, a + 2, g)
            wait_scatter(1)
            issue_loads(1, ib, a + 3, g)
            return 0
        lax.fori_loop(0, npair - 1, _pair, 0)

        a = 2 * npair - 2
        wait_loads(0)
        compute(0)
        issue_scatter(0, ib, a)
        wait_loads(1)
        compute(1)
        issue_scatter(1, ib, a + 1)
        if g + 1 < NG:
            wait_idx(g + 1)
            nb = (g + 1) % 2
            wait_scatter(0)
            issue_loads(0, nb, 0, g + 1)
            wait_scatter(1)
            issue_loads(1, nb, 1, g + 1)
        else:
            wait_scatter(0)
            wait_scatter(1)

    plsc.subcore_barrier()

    pltpu.sync_copy(hacc_sh.at[pl.ds(sid * ROWS_PT, ROWS_PT)],
                    out_hbm.at[cid, pl.ds(sid * ROWS_PT, ROWS_PT)])


def _msg_sc(x, src_t, dst_t, gate):
    mesh = plsc.VectorSubcoreMesh(core_axis_name="c", subcore_axis_name="s")
    f = pl.kernel(
        _msg_body,
        mesh=mesh,
        out_type=jax.ShapeDtypeStruct((NC, N_PAD, D_FEAT), jnp.float32),
        scratch_types=[
            pltpu.VMEM((GSZ, CHUNK), jnp.int32),
            pltpu.VMEM((GSZ, CHUNK), jnp.int32),
            pltpu.VMEM((GSZ, CHUNK), jnp.int32),
            pltpu.VMEM((GSZ, CHUNK), jnp.int32),
            pltpu.VMEM((CHUNK, D_FEAT), jnp.float32),
            pltpu.VMEM((CHUNK, D_FEAT), jnp.float32),
            pltpu.VMEM((CHUNK, D_FEAT), jnp.float32),
            pltpu.VMEM((CHUNK, D_FEAT), jnp.float32),
            pltpu.VMEM_SHARED((N_PAD, D_FEAT), jnp.float32),
            pltpu.SemaphoreType.DMA,
            pltpu.SemaphoreType.DMA,
            pltpu.SemaphoreType.DMA,
            pltpu.SemaphoreType.DMA,
            pltpu.SemaphoreType.DMA,
            pltpu.SemaphoreType.DMA,
            pltpu.SemaphoreType.DMA,
        ],
    )
    return f(x, src_t, dst_t, gate)


def _readout_body(hp_ref, wh_ref, wo_ref, sp_ref, cw_ref, out_ref):
    i = pl.program_id(0)
    h = hp_ref[0] + hp_ref[1]
    z = jax.lax.dot_general(h, wh_ref[...], (((1,), (0,)), ((), ())),
                            preferred_element_type=jnp.float32)
    s = z / (1.0 + jnp.exp(-z))
    e_sum = jnp.sum(s * wo_ref[...])
    sp = sp_ref[0, 0, :]
    comp = jnp.float32(0.0)
    for k in range(4):
        comp += cw_ref[0, k] * jnp.sum(jnp.where(sp == k, 1.0, 0.0))

    @pl.when(i == 0)
    def _():
        out_ref[0, 0] = 0.0

    out_ref[0, 0] += e_sum + comp


def _readout_tc(hpart, w_hidden, w_out2d, species3d, comp2d):
    blk = 2048
    grid = N_PAD // blk
    return pl.pallas_call(
        _readout_body,
        grid=(grid,),
        in_specs=[
            pl.BlockSpec((NC, blk, D_FEAT), lambda i: (0, i, 0)),
            pl.BlockSpec((D_FEAT, D_FEAT), lambda i: (0, 0)),
            pl.BlockSpec((1, D_FEAT), lambda i: (0, 0)),
            pl.BlockSpec((1, 1, blk), lambda i: (i, 0, 0)),
            pl.BlockSpec((1, 8), lambda i: (0, 0)),
        ],
        out_specs=pl.BlockSpec((1, 1), lambda i: (0, 0),
                               memory_space=pltpu.SMEM),
        out_shape=jax.ShapeDtypeStruct((1, 1), jnp.float32),
    )(hpart, w_hidden, w_out2d, species3d, comp2d)


def kernel(x, edge_index, edge_attr, species, W_edge, W_hidden, w_out, comp_w):
    pad = E_PAD - N_EDGES
    ea_pad = jnp.pad(edge_attr, ((0, pad), (0, 0)))
    def _tile_idx(flat):
        c0 = flat[:CORE0_EDGES].reshape(NS, NG, G0, CHUNK)
        c1 = flat[CORE0_EDGES:].reshape(NS, NG, G1, CHUNK)
        c1 = jnp.pad(c1, ((0, 0), (0, 0), (0, G0 - G1), (0, 0)))
        return jnp.concatenate([c0, c1], axis=0)

    src_t = _tile_idx(jnp.pad(edge_index[0], (0, pad)))
    dst_t = _tile_idx(jnp.pad(edge_index[1], (0, pad)))

    gate = _gate_tc(ea_pad, W_edge)
    hpart = _msg_sc(x, src_t, dst_t, gate)

    blk = 2048
    species_pad = jnp.pad(species, (0, N_PAD - N_NODES), constant_values=4)
    total = _readout_tc(
        hpart, W_hidden,
        w_out.reshape(1, D_FEAT),
        species_pad.reshape(N_PAD // blk, 1, blk),
        jnp.pad(comp_w, (0, 4)).reshape(1, 8),
    )
    return total[0, 0]


# speedup vs baseline: 1.1244x; 1.0189x over previous
import functools

import jax
import jax.numpy as jnp
from jax import lax
from jax.experimental import pallas as pl
from jax.experimental.pallas import tpu as pltpu
from jax.experimental.pallas import tpu_sc as plsc

N_NODES = 10000
N_EDGES = 320000
D_FEAT = 128
D_EDGE = 16

NC = 2
NS = 16
NW = NC * NS
CHUNK = 64
GSZ = 30
NG = 8
G0 = 30
G1 = 10
T0 = NG * G0 * CHUNK
T1 = NG * G1 * CHUNK
CORE0_EDGES = NS * T0
E_PAD = NS * (T0 + T1)
N_PAD = 10240
ROWS_PT = N_PAD // NS
ZROWS = 128


def _gate_body(ea_ref, we_ref, out_ref):
    out_ref[...] = jax.lax.dot_general(
        ea_ref[...], we_ref[...], (((1,), (0,)), ((), ())),
        preferred_element_type=jnp.float32)


def _gate_tc(ea_pad, w_edge):
    blk = 8192
    grid = E_PAD // blk
    return pl.pallas_call(
        _gate_body,
        grid=(grid,),
        in_specs=[
            pl.BlockSpec((blk, D_EDGE), lambda i: (i, 0)),
            pl.BlockSpec((D_EDGE, D_FEAT), lambda i: (0, 0)),
        ],
        out_specs=pl.BlockSpec((blk, D_FEAT), lambda i: (i, 0)),
        out_shape=jax.ShapeDtypeStruct((E_PAD, D_FEAT), jnp.float32),
    )(ea_pad, w_edge)


def _msg_body(x_hbm, src_hbm, dst_hbm, gate_hbm, out_hbm,
              src0, dst0, src1, dst1, rows0, rows1, gbuf0, gbuf1, hacc_sh,
              semg0, semg1, seml0, seml1, sems0, sems1, semi):
    cid = lax.axis_index("c")
    sid = lax.axis_index("s")
    wid = cid * NS + sid
    gc = jnp.where(cid == 0, G0, G1)
    npair = jnp.where(cid == 0, G0 // 2, G1 // 2)
    start = jnp.where(cid == 0, sid * T0, CORE0_EDGES + sid * T1)

    idx = ((src0, dst0), (src1, dst1))
    rows = (rows0, rows1)
    gbuf = (gbuf0, gbuf1)
    semg = (semg0, semg1)
    seml = (seml0, seml1)
    sems = (sems0, sems1)

    def _zrow(r, _):
        for b in range(D_FEAT // 16):
            rows0[r, pl.ds(b * 16, 16)] = jnp.zeros((16,), jnp.float32)
        return 0
    lax.fori_loop(0, CHUNK, _zrow, 0)
    for z in range(ROWS_PT // CHUNK):
        pltpu.sync_copy(rows0,
                        hacc_sh.at[pl.ds(sid * ROWS_PT + z * CHUNK, CHUNK)])
    plsc.subcore_barrier()

    def stage_idx(g):
        b = g % 2
        pltpu.async_copy(src_hbm.at[wid, g], idx[b][0], semi)
        pltpu.async_copy(dst_hbm.at[wid, g], idx[b][1], semi)

    def wait_idx(g):
        b = g % 2
        pltpu.make_async_copy(src_hbm.at[wid, 0], idx[b][0], semi).wait()
        pltpu.make_async_copy(src_hbm.at[wid, 0], idx[b][1], semi).wait()

    def issue_loads(p, ib, cj, g):
        off = start + (g * gc + cj) * CHUNK
        pltpu.async_copy(x_hbm.at[idx[ib][0].at[cj]], rows[p], semg[p])
        pltpu.async_copy(gate_hbm.at[pl.ds(off, CHUNK)], gbuf[p], seml[p])

    def wait_loads(p):
        pltpu.make_async_copy(gate_hbm.at[pl.ds(0, CHUNK)],
                              gbuf[p], seml[p]).wait()
        pltpu.make_async_copy(x_hbm.at[pl.ds(0, CHUNK)],
                              rows[p], semg[p]).wait()

    def compute(p):
        rp, gp = rows[p], gbuf[p]

        def _edge(e, _):
            for b in range(D_FEAT // 16):
                sl = pl.ds(b * 16, 16)
                gp[e, sl] = gp[e, sl] * rp[e, sl]
            return 0
        lax.fori_loop(0, CHUNK, _edge, 0)

    def issue_scatter(p, ib, cj):
        pltpu.async_copy(gbuf[p], hacc_sh.at[idx[ib][1].at[cj]], sems[p],
                         add=True)

    def wait_scatter(p):
        pltpu.make_async_copy(gbuf[p], hacc_sh.at[pl.ds(0, CHUNK)],
                              sems[p]).wait()

    stage_idx(0)
    wait_idx(0)
    issue_loads(0, 0, 0, 0)
    issue_loads(1, 0, 1, 0)

    for g in range(NG):
        ib = g % 2
        if g + 1 < NG:
            stage_idx(g + 1)

        def _pair(i, _, ib=ib, g=g):
            a = 2 * i
            wait_loads(0)
            compute(0)
            issue_scatter(0, ib, a)
            wait_loads(1)
            compute(1)
            issue_scatter(1, ib, a + 1)
            wait_scatter(0)
            issue_loads(0, ib, a + 2, g)
            wait_scatter(1)
            issue_loads(1, ib, a + 3, g)
            return 0
        lax.fori_loop(0, npair - 1, _pair, 0)

        a = 2 * npair - 2
        wait_loads(0)
        compute(0)
        issue_scatter(0, ib, a)
        wait_loads(1)
        compute(1)
        issue_scatter(1, ib, a + 1)
        if g + 1 < NG:
            wait_idx(g + 1)
            nb = (g + 1) % 2
            wait_scatter(0)
            issue_loads(0, nb, 0, g + 1)
            wait_scatter(1)
            issue_loads(1, nb, 1, g + 1)
        else:
            wait_scatter(0)
            wait_scatter(1)

    plsc.subcore_barrier()

    pltpu.sync_copy(hacc_sh.at[pl.ds(sid * ROWS_PT, ROWS_PT)],
                    out_hbm.at[cid, pl.ds(sid * ROWS_PT, ROWS_PT)])


def _msg_sc(x, src_t, dst_t, gate):
    mesh = plsc.VectorSubcoreMesh(core_axis_name="c", subcore_axis_name="s")
    f = pl.kernel(
        _msg_body,
        mesh=mesh,
        out_type=jax.ShapeDtypeStruct((NC, N_PAD, D_FEAT), jnp.float32),
        scratch_types=[
            pltpu.VMEM((GSZ, CHUNK), jnp.int32),
            pltpu.VMEM((GSZ, CHUNK), jnp.int32),
            pltpu.VMEM((GSZ, CHUNK), jnp.int32),
            pltpu.VMEM((GSZ, CHUNK), jnp.int32),
            pltpu.VMEM((CHUNK, D_FEAT), jnp.float32),
            pltpu.VMEM((CHUNK, D_FEAT), jnp.float32),
            pltpu.VMEM((CHUNK, D_FEAT), jnp.float32),
            pltpu.VMEM((CHUNK, D_FEAT), jnp.float32),
            pltpu.VMEM_SHARED((N_PAD, D_FEAT), jnp.float32),
            pltpu.SemaphoreType.DMA,
            pltpu.SemaphoreType.DMA,
            pltpu.SemaphoreType.DMA,
            pltpu.SemaphoreType.DMA,
            pltpu.SemaphoreType.DMA,
            pltpu.SemaphoreType.DMA,
            pltpu.SemaphoreType.DMA,
        ],
    )
    return f(x, src_t, dst_t, gate)


def _readout_body(hp_ref, wh_ref, wo_ref, sp_ref, cw_ref, out_ref):
    i = pl.program_id(0)
    h = hp_ref[0] + hp_ref[1]
    z = jax.lax.dot_general(h, wh_ref[...], (((1,), (0,)), ((), ())),
                            preferred_element_type=jnp.float32)
    s = z / (1.0 + jnp.exp(-z))
    e_sum = jnp.sum(s * wo_ref[...])
    sp = sp_ref[0, 0, :]
    comp = jnp.float32(0.0)
    for k in range(4):
        comp += cw_ref[0, k] * jnp.sum(jnp.where(sp == k, 1.0, 0.0))

    @pl.when(i == 0)
    def _():
        out_ref[0, 0] = 0.0

    out_ref[0, 0] += e_sum + comp


def _readout_tc(hpart, w_hidden, w_out2d, species3d, comp2d):
    blk = 2048
    grid = N_PAD // blk
    return pl.pallas_call(
        _readout_body,
        grid=(grid,),
        in_specs=[
            pl.BlockSpec((NC, blk, D_FEAT), lambda i: (0, i, 0)),
            pl.BlockSpec((D_FEAT, D_FEAT), lambda i: (0, 0)),
            pl.BlockSpec((1, D_FEAT), lambda i: (0, 0)),
            pl.BlockSpec((1, 1, blk), lambda i: (i, 0, 0)),
            pl.BlockSpec((1, 8), lambda i: (0, 0)),
        ],
        out_specs=pl.BlockSpec((1, 1), lambda i: (0, 0),
                               memory_space=pltpu.SMEM),
        out_shape=jax.ShapeDtypeStruct((1, 1), jnp.float32),
    )(hpart, w_hidden, w_out2d, species3d, comp2d)


def kernel(x, edge_index, edge_attr, species, W_edge, W_hidden, w_out, comp_w):
    pad = E_PAD - N_EDGES
    ea_pad = jnp.pad(edge_attr, ((0, pad), (0, 0)))
    def _tile_idx(flat):
        c0 = flat[:CORE0_EDGES].reshape(NS, NG, G0, CHUNK)
        c1 = flat[CORE0_EDGES:].reshape(NS, NG, G1, CHUNK)
        c1 = jnp.pad(c1, ((0, 0), (0, 0), (0, G0 - G1), (0, 0)))
        return jnp.concatenate([c0, c1], axis=0)

    src_t = _tile_idx(jnp.pad(edge_index[0], (0, pad)))
    dst_t = _tile_idx(jnp.pad(edge_index[1], (0, pad)))

    gate = _gate_tc(ea_pad, W_edge)
    hpart = _msg_sc(x, src_t, dst_t, gate)

    blk = 2048
    species_pad = jnp.pad(species, (0, N_PAD - N_NODES), constant_values=4)
    total = _readout_tc(
        hpart, W_hidden,
        w_out.reshape(1, D_FEAT),
        species_pad.reshape(N_PAD // blk, 1, blk),
        jnp.pad(comp_w, (0, 4)).reshape(1, 8),
    )
    return total[0, 0]


# speedup vs baseline: 1.1262x; 1.0016x over previous
import functools

import jax
import jax.numpy as jnp
from jax import lax
from jax.experimental import pallas as pl
from jax.experimental.pallas import tpu as pltpu
from jax.experimental.pallas import tpu_sc as plsc

N_NODES = 10000
N_EDGES = 320000
D_FEAT = 128
D_EDGE = 16

NC = 2
NS = 16
NW = NC * NS
CHUNK = 64
GSZ = 30
NG = 8
G0 = 30
G1 = 10
T0 = NG * G0 * CHUNK
T1 = NG * G1 * CHUNK
CORE0_EDGES = NS * T0
E_PAD = NS * (T0 + T1)
N_PAD = 10240
ROWS_PT = N_PAD // NS
ZROWS = 128


def _gate_body(ea_ref, we_ref, out_ref):
    out_ref[...] = jax.lax.dot_general(
        ea_ref[...], we_ref[...], (((1,), (0,)), ((), ())),
        preferred_element_type=jnp.float32)


def _gate_tc(ea_pad, w_edge):
    blk = 16384
    grid = E_PAD // blk
    return pl.pallas_call(
        _gate_body,
        grid=(grid,),
        in_specs=[
            pl.BlockSpec((blk, D_EDGE), lambda i: (i, 0)),
            pl.BlockSpec((D_EDGE, D_FEAT), lambda i: (0, 0)),
        ],
        out_specs=pl.BlockSpec((blk, D_FEAT), lambda i: (i, 0)),
        out_shape=jax.ShapeDtypeStruct((E_PAD, D_FEAT), jnp.float32),
    )(ea_pad, w_edge)


def _msg_body(x_hbm, src_hbm, dst_hbm, gate_hbm, out_hbm,
              src0, dst0, src1, dst1, rows0, rows1, gbuf0, gbuf1, hacc_sh,
              semg0, semg1, seml0, seml1, sems0, sems1, semi):
    cid = lax.axis_index("c")
    sid = lax.axis_index("s")
    wid = cid * NS + sid
    gc = jnp.where(cid == 0, G0, G1)
    npair = jnp.where(cid == 0, G0 // 2, G1 // 2)
    start = jnp.where(cid == 0, sid * T0, CORE0_EDGES + sid * T1)

    idx = ((src0, dst0), (src1, dst1))
    rows = (rows0, rows1)
    gbuf = (gbuf0, gbuf1)
    semg = (semg0, semg1)
    seml = (seml0, seml1)
    sems = (sems0, sems1)

    def _zrow(r, _):
        for b in range(D_FEAT // 16):
            rows0[r, pl.ds(b * 16, 16)] = jnp.zeros((16,), jnp.float32)
        return 0
    lax.fori_loop(0, CHUNK, _zrow, 0)
    for z in range(ROWS_PT // CHUNK):
        pltpu.sync_copy(rows0,
                        hacc_sh.at[pl.ds(sid * ROWS_PT + z * CHUNK, CHUNK)])
    plsc.subcore_barrier()

    def stage_idx(g):
        b = g % 2
        pltpu.async_copy(src_hbm.at[wid, g], idx[b][0], semi)
        pltpu.async_copy(dst_hbm.at[wid, g], idx[b][1], semi)

    def wait_idx(g):
        b = g % 2
        pltpu.make_async_copy(src_hbm.at[wid, 0], idx[b][0], semi).wait()
        pltpu.make_async_copy(src_hbm.at[wid, 0], idx[b][1], semi).wait()

    def issue_loads(p, ib, cj, g):
        off = start + (g * gc + cj) * CHUNK
        pltpu.async_copy(x_hbm.at[idx[ib][0].at[cj]], rows[p], semg[p])
        pltpu.async_copy(gate_hbm.at[pl.ds(off, CHUNK)], gbuf[p], seml[p])

    def wait_loads(p):
        pltpu.make_async_copy(gate_hbm.at[pl.ds(0, CHUNK)],
                              gbuf[p], seml[p]).wait()
        pltpu.make_async_copy(x_hbm.at[pl.ds(0, CHUNK)],
                              rows[p], semg[p]).wait()

    def compute(p):
        rp, gp = rows[p], gbuf[p]

        def _edge(e, _):
            for b in range(D_FEAT // 16):
                sl = pl.ds(b * 16, 16)
                gp[e, sl] = gp[e, sl] * rp[e, sl]
            return 0
        lax.fori_loop(0, CHUNK, _edge, 0)

    def issue_scatter(p, ib, cj):
        pltpu.async_copy(gbuf[p], hacc_sh.at[idx[ib][1].at[cj]], sems[p],
                         add=True)

    def wait_scatter(p):
        pltpu.make_async_copy(gbuf[p], hacc_sh.at[pl.ds(0, CHUNK)],
                              sems[p]).wait()

    stage_idx(0)
    wait_idx(0)
    issue_loads(0, 0, 0, 0)
    issue_loads(1, 0, 1, 0)

    for g in range(NG):
        ib = g % 2
        if g + 1 < NG:
            stage_idx(g + 1)

        def _pair(i, _, ib=ib, g=g):
            a = 2 * i
            wait_loads(0)
            compute(0)
            issue_scatter(0, ib, a)
            wait_loads(1)
            compute(1)
            issue_scatter(1, ib, a + 1)
            wait_scatter(0)
            issue_loads(0, ib, a + 2, g)
            wait_scatter(1)
            issue_loads(1, ib, a + 3, g)
            return 0
        lax.fori_loop(0, npair - 1, _pair, 0)

        a = 2 * npair - 2
        wait_loads(0)
        compute(0)
        issue_scatter(0, ib, a)
        wait_loads(1)
        compute(1)
        issue_scatter(1, ib, a + 1)
        if g + 1 < NG:
            wait_idx(g + 1)
            nb = (g + 1) % 2
            wait_scatter(0)
            issue_loads(0, nb, 0, g + 1)
            wait_scatter(1)
            issue_loads(1, nb, 1, g + 1)
        else:
            wait_scatter(0)
            wait_scatter(1)

    plsc.subcore_barrier()

    pltpu.sync_copy(hacc_sh.at[pl.ds(sid * ROWS_PT, ROWS_PT)],
                    out_hbm.at[cid, pl.ds(sid * ROWS_PT, ROWS_PT)])


def _msg_sc(x, src_t, dst_t, gate):
    mesh = plsc.VectorSubcoreMesh(core_axis_name="c", subcore_axis_name="s")
    f = pl.kernel(
        _msg_body,
        mesh=mesh,
        out_type=jax.ShapeDtypeStruct((NC, N_PAD, D_FEAT), jnp.float32),
        scratch_types=[
            pltpu.VMEM((GSZ, CHUNK), jnp.int32),
            pltpu.VMEM((GSZ, CHUNK), jnp.int32),
            pltpu.VMEM((GSZ, CHUNK), jnp.int32),
            pltpu.VMEM((GSZ, CHUNK), jnp.int32),
            pltpu.VMEM((CHUNK, D_FEAT), jnp.float32),
            pltpu.VMEM((CHUNK, D_FEAT), jnp.float32),
            pltpu.VMEM((CHUNK, D_FEAT), jnp.float32),
            pltpu.VMEM((CHUNK, D_FEAT), jnp.float32),
            pltpu.VMEM_SHARED((N_PAD, D_FEAT), jnp.float32),
            pltpu.SemaphoreType.DMA,
            pltpu.SemaphoreType.DMA,
            pltpu.SemaphoreType.DMA,
            pltpu.SemaphoreType.DMA,
            pltpu.SemaphoreType.DMA,
            pltpu.SemaphoreType.DMA,
            pltpu.SemaphoreType.DMA,
        ],
    )
    return f(x, src_t, dst_t, gate)


def _readout_body(hp_ref, wh_ref, wo_ref, sp_ref, cw_ref, out_ref):
    i = pl.program_id(0)
    h = hp_ref[0] + hp_ref[1]
    z = jax.lax.dot_general(h, wh_ref[...], (((1,), (0,)), ((), ())),
                            preferred_element_type=jnp.float32)
    s = z / (1.0 + jnp.exp(-z))
    e_sum = jnp.sum(s * wo_ref[...])
    sp = sp_ref[0, 0, :]
    comp = jnp.float32(0.0)
    for k in range(4):
        comp += cw_ref[0, k] * jnp.sum(jnp.where(sp == k, 1.0, 0.0))

    @pl.when(i == 0)
    def _():
        out_ref[0, 0] = 0.0

    out_ref[0, 0] += e_sum + comp


def _readout_tc(hpart, w_hidden, w_out2d, species3d, comp2d):
    blk = 2048
    grid = N_PAD // blk
    return pl.pallas_call(
        _readout_body,
        grid=(grid,),
        in_specs=[
            pl.BlockSpec((NC, blk, D_FEAT), lambda i: (0, i, 0)),
            pl.BlockSpec((D_FEAT, D_FEAT), lambda i: (0, 0)),
            pl.BlockSpec((1, D_FEAT), lambda i: (0, 0)),
            pl.BlockSpec((1, 1, blk), lambda i: (i, 0, 0)),
            pl.BlockSpec((1, 8), lambda i: (0, 0)),
        ],
        out_specs=pl.BlockSpec((1, 1), lambda i: (0, 0),
                               memory_space=pltpu.SMEM),
        out_shape=jax.ShapeDtypeStruct((1, 1), jnp.float32),
    )(hpart, w_hidden, w_out2d, species3d, comp2d)


def kernel(x, edge_index, edge_attr, species, W_edge, W_hidden, w_out, comp_w):
    pad = E_PAD - N_EDGES
    ea_pad = jnp.pad(edge_attr, ((0, pad), (0, 0)))
    def _tile_idx(flat):
        c0 = flat[:CORE0_EDGES].reshape(NS, NG, G0, CHUNK)
        c1 = flat[CORE0_EDGES:].reshape(NS, NG, G1, CHUNK)
        c1 = jnp.pad(c1, ((0, 0), (0, 0), (0, G0 - G1), (0, 0)))
        return jnp.concatenate([c0, c1], axis=0)

    src_t = _tile_idx(jnp.pad(edge_index[0], (0, pad)))
    dst_t = _tile_idx(jnp.pad(edge_index[1], (0, pad)))

    gate = _gate_tc(ea_pad, W_edge)
    hpart = _msg_sc(x, src_t, dst_t, gate)

    blk = 2048
    species_pad = jnp.pad(species, (0, N_PAD - N_NODES), constant_values=4)
    total = _readout_tc(
        hpart, W_hidden,
        w_out.reshape(1, D_FEAT),
        species_pad.reshape(N_PAD // blk, 1, blk),
        jnp.pad(comp_w, (0, 4)).reshape(1, 8),
    )
    return total[0, 0]
